# baseline (speedup 1.0000x reference)
import functools

import jax
import jax.numpy as jnp
from jax import lax
from jax.experimental import pallas as pl
from jax.experimental.pallas import tpu as pltpu
from jax.experimental.pallas import tpu_sc as plsc

B, L, D = 4, 2048, 64
K_MAX_OUT = 64
N_ROWS = 2 * B * L
PACK_BLK = 8192
N_TILES = 32
TILES_PER_BATCH = N_TILES // B
Q_PER_TILE = L // TILES_PER_BATCH
N_CHUNKS = L // 16


def _sig(x, w):
    bits = jnp.where(x > 0, 1.0, 0.0)
    h = jnp.dot(bits, w, preferred_element_type=jnp.float32)
    h0 = h[:, 0:1].astype(jnp.int32)
    h1 = h[:, 1:2].astype(jnp.int32)
    h2 = h[:, 2:3].astype(jnp.int32)
    h3 = h[:, 3:4].astype(jnp.int32)
    return h0 | (h1 << 16), h2 | (h3 << 16)


def _pack_body(w_ref, q_ref, k_ref, klo_ref, khi_ref, lor_ref, hir_ref):
    w = w_ref[...]
    klo, khi = _sig(k_ref[...], w)
    klo_ref[...] = klo
    khi_ref[...] = khi
    qlo, qhi = _sig(q_ref[...], w)
    lor_ref[...] = jnp.broadcast_to(qlo, (PACK_BLK, 16))
    hir_ref[...] = jnp.broadcast_to(qhi, (PACK_BLK, 16))


_pack = pl.pallas_call(
    _pack_body,
    grid=(B * L // PACK_BLK,),
    in_specs=[
        pl.BlockSpec((D, 128), lambda g: (0, 0)),
        pl.BlockSpec((PACK_BLK, D), lambda g: (g, 0)),
        pl.BlockSpec((PACK_BLK, D), lambda g: (g, 0)),
    ],
    out_specs=[
        pl.BlockSpec((PACK_BLK, 1), lambda g: (g, 0)),
        pl.BlockSpec((PACK_BLK, 1), lambda g: (g, 0)),
        pl.BlockSpec((PACK_BLK, 16), lambda g: (g, 0)),
        pl.BlockSpec((PACK_BLK, 16), lambda g: (g, 0)),
    ],
    out_shape=[
        jax.ShapeDtypeStruct((B * L, 1), jnp.int32),
        jax.ShapeDtypeStruct((B * L, 1), jnp.int32),
        jax.ShapeDtypeStruct((B * L, 16), jnp.int32),
        jax.ShapeDtypeStruct((B * L, 16), jnp.int32),
    ],
)


NB = 512
CAP = 16


def _find_body(lo_hbm, hi_hbm, lor_hbm, hir_hbm, out_hbm,
               qlo_v, qhi_v, klo_v, khi_v, bcnt_v, blo_v, bhi_v, bidx_v,
               row_v, out_v):
    c = lax.axis_index("c")
    s = lax.axis_index("s")
    wid = s * 2 + c
    b = wid // TILES_PER_BATCH
    j = wid % TILES_PER_BATCH
    qbase = b * L + j * Q_PER_TILE
    kbase = b * L

    pltpu.sync_copy(lor_hbm.at[pl.ds(qbase, Q_PER_TILE), :], qlo_v)
    pltpu.sync_copy(hir_hbm.at[pl.ds(qbase, Q_PER_TILE), :], qhi_v)
    pltpu.sync_copy(lo_hbm.at[pl.ds(kbase, L)], klo_v)
    pltpu.sync_copy(hi_hbm.at[pl.ds(kbase, L)], khi_v)

    lane = lax.iota(jnp.int32, 16)
    neg1 = jnp.full((16,), -1.0, jnp.float32)
    zero16 = jnp.zeros((16,), jnp.int32)

    def zero_cnt(t, carry):
        bcnt_v[pl.ds(t * 16, 16)] = zero16
        return carry
    lax.fori_loop(0, NB // 16, zero_cnt, jnp.int32(0))

    def build_chunk(ci, ovf):
        klo = klo_v[pl.ds(ci * 16, 16)]
        khi = khi_v[pl.ds(ci * 16, 16)]
        h = klo & (NB - 1)
        rank, lastm = plsc.scan_count(h)
        base = plsc.load_gather(bcnt_v, [h])
        pos = base + rank - 1
        ok = pos < CAP
        dest = h * CAP + jnp.minimum(pos, CAP - 1)
        plsc.store_scatter(blo_v, [dest], klo, mask=ok)
        plsc.store_scatter(bhi_v, [dest], khi, mask=ok)
        plsc.store_scatter(bidx_v, [dest], (lane + ci * 16).astype(jnp.float32),
                           mask=ok)
        plsc.addupdate_scatter(bcnt_v, [h], rank, mask=lastm)
        return ovf | jnp.where(ok, 0, 1)
    ovf = lax.fori_loop(0, N_CHUNKS, build_chunk, zero16)
    n_ovf = plsc.all_reduce_population_count(ovf > 0)[0]

    @plsc.parallel_loop(0, Q_PER_TILE * 4 + 1, unroll=8)
    def _init_out(t):
        out_v[pl.ds(t * 16, 16)] = neg1

    capv = jnp.full((16,), CAP, jnp.int32)

    @plsc.parallel_loop(0, Q_PER_TILE, unroll=8)
    def _probe(i):
        qlo = qlo_v[i, :]
        qhi = qhi_v[i, :]
        qh = qlo & (NB - 1)
        nv = plsc.load_gather(bcnt_v, [qh])
        bb = qh * CAP + lane
        blo = plsc.load_gather(blo_v, [bb])
        bhi = plsc.load_gather(bhi_v, [bb])
        bidx = plsc.load_gather(bidx_v, [bb])
        m0 = (blo == qlo) & (bhi == qhi) & (lane < nv) & (nv <= capv)
        plsc.store_compressed(out_v.at[pl.ds(i * K_MAX_OUT, 16)], bidx, mask=m0)

    @pl.when(n_ovf > 0)
    def _overflow_pass():
        def per_query(i, carry):
            qlo = qlo_v[i, :]
            qhi = qhi_v[i, :]
            qh = qlo & (NB - 1)
            n = plsc.load_gather(bcnt_v, [qh])[0]

            def full_scan():
                for t in range(5):
                    row_v[pl.ds(t * 16, 16)] = neg1

                def scan_chunk(ci, cnt):
                    klo = klo_v[pl.ds(ci * 16, 16)]
                    khi = khi_v[pl.ds(ci * 16, 16)]
                    m = jnp.logical_and(klo == qlo, khi == qhi)
                    nm = plsc.all_reduce_population_count(m)[0]

                    def matched(cn):
                        fidx = (lane + ci * 16).astype(jnp.float32)
                        off = jnp.minimum(cn, K_MAX_OUT)
                        plsc.store_compressed(row_v.at[pl.ds(off, 16)],
                                              fidx, mask=m)
                        return cn + nm

                    return lax.cond(nm > 0, matched, lambda cn: cn, cnt)

                lax.fori_loop(0, N_CHUNKS, scan_chunk, jnp.int32(0))
                for t in range(4):
                    out_v[pl.ds(i * K_MAX_OUT + t * 16, 16)] = \
                        row_v[pl.ds(t * 16, 16)]

            lax.cond(n > CAP, full_scan, lambda: None)
            return carry

        lax.fori_loop(0, Q_PER_TILE, per_query, jnp.int32(0))
    pltpu.sync_copy(out_v.at[pl.ds(0, Q_PER_TILE * K_MAX_OUT)],
                    out_hbm.at[pl.ds(qbase * K_MAX_OUT, Q_PER_TILE * K_MAX_OUT)])


@functools.cache
def _build_find():
    return pl.kernel(
        _find_body,
        out_type=jax.ShapeDtypeStruct((B * L * K_MAX_OUT,), jnp.float32),
        mesh=plsc.VectorSubcoreMesh(core_axis_name="c", subcore_axis_name="s"),
        compiler_params=pltpu.CompilerParams(needs_layout_passes=False),
        scratch_types=[
            pltpu.VMEM((Q_PER_TILE, 16), jnp.int32),
            pltpu.VMEM((Q_PER_TILE, 16), jnp.int32),
            pltpu.VMEM((L,), jnp.int32),
            pltpu.VMEM((L,), jnp.int32),
            pltpu.VMEM((NB,), jnp.int32),
            pltpu.VMEM((NB * CAP,), jnp.int32),
            pltpu.VMEM((NB * CAP,), jnp.int32),
            pltpu.VMEM((NB * CAP,), jnp.float32),
            pltpu.VMEM((K_MAX_OUT + 16,), jnp.float32),
            pltpu.VMEM((Q_PER_TILE * K_MAX_OUT + 16,), jnp.float32),
        ],
    )


import numpy as _np

_W_NP = _np.zeros((D, 128), _np.float32)
for _d in range(D):
    _W_NP[_d, _d // 16] = float(1 << (_d % 16))


@jax.jit
def kernel(query_up, key_up, head_idx):
    klo, khi, lor, hir = _pack(jnp.asarray(_W_NP), query_up.reshape(B * L, D),
                               key_up.reshape(B * L, D))
    out = _build_find()(klo.reshape(B * L), khi.reshape(B * L), lor, hir)
    return out.reshape(B, L, K_MAX_OUT)


# speedup vs baseline: 1.0031x; 1.0031x over previous
import functools

import jax
import jax.numpy as jnp
from jax import lax
from jax.experimental import pallas as pl
from jax.experimental.pallas import tpu as pltpu
from jax.experimental.pallas import tpu_sc as plsc

B, L, D = 4, 2048, 64
K_MAX_OUT = 64
N_ROWS = 2 * B * L
PACK_BLK = 2048
N_TILES = 32
TILES_PER_BATCH = N_TILES // B
Q_PER_TILE = L // TILES_PER_BATCH
N_CHUNKS = L // 16


def _sig(x, w):
    bits = jnp.where(x > 0, 1.0, 0.0)
    h = jnp.dot(bits, w, preferred_element_type=jnp.float32)
    h0 = h[:, 0:1].astype(jnp.int32)
    h1 = h[:, 1:2].astype(jnp.int32)
    h2 = h[:, 2:3].astype(jnp.int32)
    h3 = h[:, 3:4].astype(jnp.int32)
    return h0 | (h1 << 16), h2 | (h3 << 16)


def _pack_body(w_ref, q_ref, k_ref, klo_ref, khi_ref, lor_ref, hir_ref):
    w = w_ref[...]
    klo, khi = _sig(k_ref[...], w)
    klo_ref[...] = klo
    khi_ref[...] = khi
    qlo, qhi = _sig(q_ref[...], w)
    lor_ref[...] = jnp.broadcast_to(qlo, (PACK_BLK, 16))
    hir_ref[...] = jnp.broadcast_to(qhi, (PACK_BLK, 16))


_pack = pl.pallas_call(
    _pack_body,
    grid=(B * L // PACK_BLK,),
    in_specs=[
        pl.BlockSpec((D, 128), lambda g: (0, 0)),
        pl.BlockSpec((PACK_BLK, D), lambda g: (g, 0)),
        pl.BlockSpec((PACK_BLK, D), lambda g: (g, 0)),
    ],
    out_specs=[
        pl.BlockSpec((PACK_BLK, 1), lambda g: (g, 0)),
        pl.BlockSpec((PACK_BLK, 1), lambda g: (g, 0)),
        pl.BlockSpec((PACK_BLK, 16), lambda g: (g, 0)),
        pl.BlockSpec((PACK_BLK, 16), lambda g: (g, 0)),
    ],
    out_shape=[
        jax.ShapeDtypeStruct((B * L, 1), jnp.int32),
        jax.ShapeDtypeStruct((B * L, 1), jnp.int32),
        jax.ShapeDtypeStruct((B * L, 16), jnp.int32),
        jax.ShapeDtypeStruct((B * L, 16), jnp.int32),
    ],
)


NB = 512
CAP = 16


def _find_body(lo_hbm, hi_hbm, lor_hbm, hir_hbm, out_hbm,
               qlo_v, qhi_v, klo_v, khi_v, bcnt_v, blo_v, bhi_v, bidx_v,
               row_v, out_v):
    c = lax.axis_index("c")
    s = lax.axis_index("s")
    wid = s * 2 + c
    b = wid // TILES_PER_BATCH
    j = wid % TILES_PER_BATCH
    qbase = b * L + j * Q_PER_TILE
    kbase = b * L

    pltpu.sync_copy(lor_hbm.at[pl.ds(qbase, Q_PER_TILE), :], qlo_v)
    pltpu.sync_copy(hir_hbm.at[pl.ds(qbase, Q_PER_TILE), :], qhi_v)
    pltpu.sync_copy(lo_hbm.at[pl.ds(kbase, L)], klo_v)
    pltpu.sync_copy(hi_hbm.at[pl.ds(kbase, L)], khi_v)

    lane = lax.iota(jnp.int32, 16)
    neg1 = jnp.full((16,), -1.0, jnp.float32)
    zero16 = jnp.zeros((16,), jnp.int32)

    def zero_cnt(t, carry):
        bcnt_v[pl.ds(t * 16, 16)] = zero16
        return carry
    lax.fori_loop(0, NB // 16, zero_cnt, jnp.int32(0))

    def build_chunk(ci, ovf):
        klo = klo_v[pl.ds(ci * 16, 16)]
        khi = khi_v[pl.ds(ci * 16, 16)]
        h = klo & (NB - 1)
        rank, lastm = plsc.scan_count(h)
        base = plsc.load_gather(bcnt_v, [h])
        pos = base + rank - 1
        ok = pos < CAP
        dest = h * CAP + jnp.minimum(pos, CAP - 1)
        plsc.store_scatter(blo_v, [dest], klo, mask=ok)
        plsc.store_scatter(bhi_v, [dest], khi, mask=ok)
        plsc.store_scatter(bidx_v, [dest], (lane + ci * 16).astype(jnp.float32),
                           mask=ok)
        plsc.addupdate_scatter(bcnt_v, [h], rank, mask=lastm)
        return ovf | jnp.where(ok, 0, 1)
    ovf = lax.fori_loop(0, N_CHUNKS, build_chunk, zero16)
    n_ovf = plsc.all_reduce_population_count(ovf > 0)[0]

    @plsc.parallel_loop(0, Q_PER_TILE * 4 + 1, unroll=8)
    def _init_out(t):
        out_v[pl.ds(t * 16, 16)] = neg1

    capv = jnp.full((16,), CAP, jnp.int32)

    @plsc.parallel_loop(0, Q_PER_TILE, unroll=8)
    def _probe(i):
        qlo = qlo_v[i, :]
        qhi = qhi_v[i, :]
        qh = qlo & (NB - 1)
        nv = plsc.load_gather(bcnt_v, [qh])
        bb = qh * CAP + lane
        blo = plsc.load_gather(blo_v, [bb])
        bhi = plsc.load_gather(bhi_v, [bb])
        bidx = plsc.load_gather(bidx_v, [bb])
        m0 = (blo == qlo) & (bhi == qhi) & (lane < nv) & (nv <= capv)
        plsc.store_compressed(out_v.at[pl.ds(i * K_MAX_OUT, 16)], bidx, mask=m0)

    @pl.when(n_ovf > 0)
    def _overflow_pass():
        def per_query(i, carry):
            qlo = qlo_v[i, :]
            qhi = qhi_v[i, :]
            qh = qlo & (NB - 1)
            n = plsc.load_gather(bcnt_v, [qh])[0]

            def full_scan():
                for t in range(5):
                    row_v[pl.ds(t * 16, 16)] = neg1

                def scan_chunk(ci, cnt):
                    klo = klo_v[pl.ds(ci * 16, 16)]
                    khi = khi_v[pl.ds(ci * 16, 16)]
                    m = jnp.logical_and(klo == qlo, khi == qhi)
                    nm = plsc.all_reduce_population_count(m)[0]

                    def matched(cn):
                        fidx = (lane + ci * 16).astype(jnp.float32)
                        off = jnp.minimum(cn, K_MAX_OUT)
                        plsc.store_compressed(row_v.at[pl.ds(off, 16)],
                                              fidx, mask=m)
                        return cn + nm

                    return lax.cond(nm > 0, matched, lambda cn: cn, cnt)

                lax.fori_loop(0, N_CHUNKS, scan_chunk, jnp.int32(0))
                for t in range(4):
                    out_v[pl.ds(i * K_MAX_OUT + t * 16, 16)] = \
                        row_v[pl.ds(t * 16, 16)]

            lax.cond(n > CAP, full_scan, lambda: None)
            return carry

        lax.fori_loop(0, Q_PER_TILE, per_query, jnp.int32(0))
    pltpu.sync_copy(out_v.at[pl.ds(0, Q_PER_TILE * K_MAX_OUT)],
                    out_hbm.at[pl.ds(qbase * K_MAX_OUT, Q_PER_TILE * K_MAX_OUT)])


@functools.cache
def _build_find():
    return pl.kernel(
        _find_body,
        out_type=jax.ShapeDtypeStruct((B * L * K_MAX_OUT,), jnp.float32),
        mesh=plsc.VectorSubcoreMesh(core_axis_name="c", subcore_axis_name="s"),
        compiler_params=pltpu.CompilerParams(needs_layout_passes=False),
        scratch_types=[
            pltpu.VMEM((Q_PER_TILE, 16), jnp.int32),
            pltpu.VMEM((Q_PER_TILE, 16), jnp.int32),
            pltpu.VMEM((L,), jnp.int32),
            pltpu.VMEM((L,), jnp.int32),
            pltpu.VMEM((NB,), jnp.int32),
            pltpu.VMEM((NB * CAP,), jnp.int32),
            pltpu.VMEM((NB * CAP,), jnp.int32),
            pltpu.VMEM((NB * CAP,), jnp.float32),
            pltpu.VMEM((K_MAX_OUT + 16,), jnp.float32),
            pltpu.VMEM((Q_PER_TILE * K_MAX_OUT + 16,), jnp.float32),
        ],
    )


import numpy as _np

_W_NP = _np.zeros((D, 128), _np.float32)
for _d in range(D):
    _W_NP[_d, _d // 16] = float(1 << (_d % 16))


@jax.jit
def kernel(query_up, key_up, head_idx):
    klo, khi, lor, hir = _pack(jnp.asarray(_W_NP), query_up.reshape(B * L, D),
                               key_up.reshape(B * L, D))
    out = _build_find()(klo.reshape(B * L), khi.reshape(B * L), lor, hir)
    return out.reshape(B, L, K_MAX_OUT)
